# Initial kernel scaffold; baseline (speedup 1.0000x reference)
import functools

import jax
import jax.numpy as jnp
from jax.experimental import pallas as pl

B = 16384
DIM = 512
NBINS = 64
BOUND = 10.0
NLAYERS = 2
EPS = 1e-6

BB = 64


def _softplus(x):
    return jnp.maximum(x, 0.0) + jnp.log(1.0 + jnp.exp(-jnp.abs(x)))


def _prep_kernel(raw_wT, raw_hT, raw_sT, log_scale,
                 K, AX, AY, AW, AH, AD0, AD1, SCALE):
    ir = jax.lax.broadcasted_iota(jnp.float32, (NBINS, NBINS), 0)
    ic = jax.lax.broadcasted_iota(jnp.float32, (NBINS, NBINS), 1)
    L_tri = (ic <= ir).astype(jnp.float32)

    def tables(raw):
        m = jnp.max(raw, axis=0, keepdims=True)
        e = jnp.exp(raw - m)
        w = e / jnp.sum(e, axis=0, keepdims=True) * (2.0 * BOUND)
        cw = jax.lax.dot_general(L_tri, w, (((1,), (0,)), ((), ())),
                                 preferred_element_type=jnp.float32)
        return w, cw

    for l in range(NLAYERS):
        w, cw = tables(raw_wT[l])
        h, _ = tables(raw_hT[l])
        s = _softplus(raw_sT[l]) + 1e-4
        sl = pl.ds(l * NBINS, NBINS)
        neg = jnp.full((1, DIM), -BOUND, jnp.float32)
        K[sl] = jnp.concatenate(
            [jnp.full((1, DIM), -1e30, jnp.float32), cw[: NBINS - 1] - BOUND],
            axis=0).reshape(NBINS, 1, DIM)
        AX[sl] = jnp.concatenate([neg, w[: NBINS - 1]], axis=0).reshape(NBINS, 1, DIM)
        AY[sl] = jnp.concatenate([neg, h[: NBINS - 1]], axis=0).reshape(NBINS, 1, DIM)
        AW[sl] = jnp.concatenate([w[0:1], w[1:] - w[:-1]], axis=0).reshape(NBINS, 1, DIM)
        AH[sl] = jnp.concatenate([h[0:1], h[1:] - h[:-1]], axis=0).reshape(NBINS, 1, DIM)
        AD0[sl] = jnp.concatenate([s[0:1], s[1:NBINS] - s[: NBINS - 1]],
                                  axis=0).reshape(NBINS, 1, DIM)
        AD1[sl] = jnp.concatenate([s[1:2], s[2:] - s[1:NBINS]],
                                  axis=0).reshape(NBINS, 1, DIM)
    SCALE[...] = _softplus(log_scale[...]) + 1e-4


def _apply_kernel(u, tau, bias, K, AX, AY, AW, AH, AD0, AD1, SCALE, out):
    u_safe = jnp.clip(u[...], EPS, 1.0 - EPS)
    z = jnp.log(u_safe / (1.0 - u_safe))

    for l in range(NLAYERS):
        inside = (z > -BOUND) & (z < BOUND)
        zc = jnp.clip(z, -BOUND, BOUND)

        def body(k, accs, l=l, zc=zc):
            ax, ay, aw, ah, ad0, ad1 = accs
            r = l * NBINS + k
            c = zc >= K[r]
            ax = jnp.where(c, ax + AX[r], ax)
            ay = jnp.where(c, ay + AY[r], ay)
            aw = jnp.where(c, aw + AW[r], aw)
            ah = jnp.where(c, ah + AH[r], ah)
            ad0 = jnp.where(c, ad0 + AD0[r], ad0)
            ad1 = jnp.where(c, ad1 + AD1[r], ad1)
            return ax, ay, aw, ah, ad0, ad1

        zeros = jnp.zeros_like(z)
        ax, ay, aw, ah, ad0, ad1 = jax.lax.fori_loop(
            0, NBINS, body, (zeros, zeros, zeros, zeros, zeros, zeros))
        s = ah / aw
        xi = jnp.clip((zc - ax) / aw, 0.0, 1.0)
        omxi = 1.0 - xi
        num = ah * (s * xi * xi + ad0 * xi * omxi)
        den = s + (ad0 + ad1 - 2.0 * s) * xi * omxi
        y = ay + num / den
        z = jnp.where(inside, y, z)

    q = z * SCALE[...] + bias[...]
    out[...] = tau[...] * q


@jax.jit
def kernel(u, tau, log_scale, bias, raw_w, raw_h, raw_s):
    raw_wT = jnp.transpose(raw_w, (0, 2, 1))
    raw_hT = jnp.transpose(raw_h, (0, 2, 1))
    raw_sT = jnp.transpose(raw_s, (0, 2, 1))
    ls = log_scale.reshape(1, DIM)
    tab_shape = jax.ShapeDtypeStruct((NLAYERS * NBINS, 1, DIM), jnp.float32)
    tabs = pl.pallas_call(
        _prep_kernel,
        out_shape=(tab_shape,) * 7 + (jax.ShapeDtypeStruct((1, DIM), jnp.float32),),
    )(raw_wT, raw_hT, raw_sT, ls)
    K, AX, AY, AW, AH, AD0, AD1, SCALE = tabs

    full = lambda shape: pl.BlockSpec(shape, lambda i: (0,) * len(shape))
    grid = B // BB
    tab_spec = full((NLAYERS * NBINS, 1, DIM))
    out = pl.pallas_call(
        _apply_kernel,
        grid=(grid,),
        in_specs=[
            pl.BlockSpec((BB, DIM), lambda i: (i, 0)),
            pl.BlockSpec((BB, 1), lambda i: (i, 0)),
            full((1, DIM)),
            tab_spec, tab_spec, tab_spec, tab_spec, tab_spec, tab_spec, tab_spec,
            full((1, DIM)),
        ],
        out_specs=pl.BlockSpec((BB, DIM), lambda i: (i, 0)),
        out_shape=jax.ShapeDtypeStruct((B, DIM), jnp.float32),
    )(u, tau, bias.reshape(1, DIM), K, AX, AY, AW, AH, AD0, AD1, SCALE)
    return out


# speedup vs baseline: 394.5782x; 394.5782x over previous
import functools

import jax
import jax.numpy as jnp
from jax.experimental import pallas as pl

B = 16384
DIM = 512
NBINS = 64
BOUND = 10.0
NLAYERS = 2
EPS = 1e-6

BB = 64


def _softplus(x):
    return jnp.maximum(x, 0.0) + jnp.log(1.0 + jnp.exp(-jnp.abs(x)))


def _prep_kernel(raw_wT, raw_hT, raw_sT, log_scale,
                 K, AX, AY, AW, AH, AD0, AD1, SCALE):
    ir = jax.lax.broadcasted_iota(jnp.int32, (NBINS, NBINS), 0)
    ic = jax.lax.broadcasted_iota(jnp.int32, (NBINS, NBINS), 1)
    L_tri = (ic <= ir).astype(jnp.float32)

    def tables(raw):
        m = jnp.max(raw, axis=0, keepdims=True)
        e = jnp.exp(raw - m)
        w = e / jnp.sum(e, axis=0, keepdims=True) * (2.0 * BOUND)
        cw = jax.lax.dot_general(L_tri, w, (((1,), (0,)), ((), ())),
                                 preferred_element_type=jnp.float32)
        return w, cw

    for l in range(NLAYERS):
        w, cw = tables(raw_wT[l])
        h, _ = tables(raw_hT[l])
        s = _softplus(raw_sT[l]) + 1e-4
        sl = pl.ds(l * NBINS, NBINS)
        neg = jnp.full((1, DIM), -BOUND, jnp.float32)
        K[sl] = jnp.concatenate(
            [jnp.full((1, DIM), -1e30, jnp.float32), cw[: NBINS - 1] - BOUND],
            axis=0).reshape(NBINS, 1, DIM)
        AX[sl] = jnp.concatenate([neg, w[: NBINS - 1]], axis=0).reshape(NBINS, 1, DIM)
        AY[sl] = jnp.concatenate([neg, h[: NBINS - 1]], axis=0).reshape(NBINS, 1, DIM)
        AW[sl] = jnp.concatenate([w[0:1], w[1:] - w[:-1]], axis=0).reshape(NBINS, 1, DIM)
        AH[sl] = jnp.concatenate([h[0:1], h[1:] - h[:-1]], axis=0).reshape(NBINS, 1, DIM)
        AD0[sl] = jnp.concatenate([s[0:1], s[1:NBINS] - s[: NBINS - 1]],
                                  axis=0).reshape(NBINS, 1, DIM)
        AD1[sl] = jnp.concatenate([s[1:2], s[2:] - s[1:NBINS]],
                                  axis=0).reshape(NBINS, 1, DIM)
    SCALE[...] = _softplus(log_scale[...]) + 1e-4


def _apply_kernel(u, tau, bias, K, AX, AY, AW, AH, AD0, AD1, SCALE, out):
    u_safe = jnp.clip(u[...], EPS, 1.0 - EPS)
    z = jnp.log(u_safe / (1.0 - u_safe))

    for l in range(NLAYERS):
        inside = (z > -BOUND) & (z < BOUND)
        zc = jnp.clip(z, -BOUND, BOUND)

        def body(k, accs, l=l, zc=zc):
            ax, ay, aw, ah, ad0, ad1 = accs
            r = l * NBINS + k
            c = zc >= K[r]
            ax = jnp.where(c, ax + AX[r], ax)
            ay = jnp.where(c, ay + AY[r], ay)
            aw = jnp.where(c, aw + AW[r], aw)
            ah = jnp.where(c, ah + AH[r], ah)
            ad0 = jnp.where(c, ad0 + AD0[r], ad0)
            ad1 = jnp.where(c, ad1 + AD1[r], ad1)
            return ax, ay, aw, ah, ad0, ad1

        zeros = jnp.zeros_like(z)
        ax, ay, aw, ah, ad0, ad1 = jax.lax.fori_loop(
            0, NBINS, body, (zeros, zeros, zeros, zeros, zeros, zeros))
        s = ah / aw
        xi = jnp.clip((zc - ax) / aw, 0.0, 1.0)
        omxi = 1.0 - xi
        num = ah * (s * xi * xi + ad0 * xi * omxi)
        den = s + (ad0 + ad1 - 2.0 * s) * xi * omxi
        y = ay + num / den
        z = jnp.where(inside, y, z)

    q = z * SCALE[...] + bias[...]
    out[...] = tau[...] * q


@jax.jit
def kernel(u, tau, log_scale, bias, raw_w, raw_h, raw_s):
    raw_wT = jnp.transpose(raw_w, (0, 2, 1))
    raw_hT = jnp.transpose(raw_h, (0, 2, 1))
    raw_sT = jnp.transpose(raw_s, (0, 2, 1))
    ls = log_scale.reshape(1, DIM)
    tab_shape = jax.ShapeDtypeStruct((NLAYERS * NBINS, 1, DIM), jnp.float32)
    tabs = pl.pallas_call(
        _prep_kernel,
        out_shape=(tab_shape,) * 7 + (jax.ShapeDtypeStruct((1, DIM), jnp.float32),),
    )(raw_wT, raw_hT, raw_sT, ls)
    K, AX, AY, AW, AH, AD0, AD1, SCALE = tabs

    full = lambda shape: pl.BlockSpec(shape, lambda i: (0,) * len(shape))
    grid = B // BB
    tab_spec = full((NLAYERS * NBINS, 1, DIM))
    out = pl.pallas_call(
        _apply_kernel,
        grid=(grid,),
        in_specs=[
            pl.BlockSpec((BB, DIM), lambda i: (i, 0)),
            pl.BlockSpec((BB, 1), lambda i: (i, 0)),
            full((1, DIM)),
            tab_spec, tab_spec, tab_spec, tab_spec, tab_spec, tab_spec, tab_spec,
            full((1, DIM)),
        ],
        out_specs=pl.BlockSpec((BB, DIM), lambda i: (i, 0)),
        out_shape=jax.ShapeDtypeStruct((B, DIM), jnp.float32),
    )(u, tau, bias.reshape(1, DIM), K, AX, AY, AW, AH, AD0, AD1, SCALE)
    return out


# speedup vs baseline: 885.3510x; 2.2438x over previous
import jax
import jax.numpy as jnp
from jax import lax
from jax.experimental import pallas as pl
from jax.experimental.pallas import tpu as pltpu
from jax.experimental.pallas import tpu_sc as plsc

B = 16384
DIM = 512
NBINS = 64
BOUND = 10.0
NLAYERS = 2
EPS = 1e-6

NC = 2
NS = 16
NW = NC * NS
DPW = DIM // NW
NBCH = 1024
NCHUNK = B // NBCH
NFIELD = 6
FSTRIDE = NBINS * DPW
LSTRIDE = NFIELD * FSTRIDE
TABW = NLAYERS * LSTRIDE

BBZ = 512


def _softplus(x):
    return jnp.maximum(x, 0.0) + jnp.log(1.0 + jnp.exp(-jnp.abs(x)))


def _prep_kernel(raw_wT, raw_hT, raw_sT, log_scale, TABF, SCALE):
    ir = lax.broadcasted_iota(jnp.int32, (NBINS, NBINS), 0)
    ic = lax.broadcasted_iota(jnp.int32, (NBINS, NBINS), 1)
    L_tri = (ic <= ir).astype(jnp.float32)

    def tables(raw):
        m = jnp.max(raw, axis=0, keepdims=True)
        e = jnp.exp(raw - m)
        w = e / jnp.sum(e, axis=0, keepdims=True) * (2.0 * BOUND)
        cw = lax.dot_general(L_tri, w, (((1,), (0,)), ((), ())),
                             preferred_element_type=jnp.float32)
        return w, cw

    neg = jnp.full((1, DIM), -BOUND, jnp.float32)
    for l in range(NLAYERS):
        w, cw = tables(raw_wT[l])
        h, ch = tables(raw_hT[l])
        s = _softplus(raw_sT[l]) + 1e-4
        TABF[l, 0] = jnp.concatenate([neg, cw[: NBINS - 1] - BOUND], axis=0)
        TABF[l, 1] = 1.0 / w
        TABF[l, 2] = jnp.concatenate([neg, ch[: NBINS - 1] - BOUND], axis=0)
        TABF[l, 3] = h
        TABF[l, 4] = s[:NBINS]
        TABF[l, 5] = s[1:]
    SCALE[...] = _softplus(log_scale[...]) + 1e-4


def _logit_kernel(u, z):
    u_safe = jnp.clip(u[...], EPS, 1.0 - EPS)
    z[...] = jnp.log(u_safe / (1.0 - u_safe))


def _rqs_vreg(z, tab_v, dl):
    for l in range(NLAYERS):
        base = l * LSTRIDE
        inside = (z > -BOUND) & (z < BOUND)
        zc = jnp.minimum(jnp.maximum(z, -BOUND), BOUND)
        cur = dl + base
        for step in (32, 16, 8, 4, 2, 1):
            cand = cur + step * DPW
            xkv = plsc.load_gather(tab_v, [cand])
            cur = jnp.where(zc >= xkv, cand, cur)
        xkb = plsc.load_gather(tab_v, [cur])
        invw = plsc.load_gather(tab_v, [cur + FSTRIDE])
        yb = plsc.load_gather(tab_v, [cur + 2 * FSTRIDE])
        h = plsc.load_gather(tab_v, [cur + 3 * FSTRIDE])
        d0 = plsc.load_gather(tab_v, [cur + 4 * FSTRIDE])
        d1 = plsc.load_gather(tab_v, [cur + 5 * FSTRIDE])
        s = h * invw
        xi = jnp.minimum(jnp.maximum((zc - xkb) * invw, 0.0), 1.0)
        omxi = 1.0 - xi
        num = (h * xi) * (s * xi + d0 * omxi)
        den = s + (d0 + d1 - 2.0 * s) * (xi * omxi)
        y = yb + num / den
        z = jnp.where(inside, y, z)
    return z


RPG = DPW * 8 // DPW
QCH = NBCH // 8


def _sc_body(z_hbm, tau_hbm, tab_hbm, scale_hbm, bias_hbm, out_hbm,
             tab_v, zbuf, obuf, taubuf, sbbuf):
    w = lax.axis_index("s") * NC + lax.axis_index("c")
    dcol = pl.multiple_of(w * DPW, DPW)
    pltpu.sync_copy(tab_hbm.at[pl.ds(pl.multiple_of(w * TABW, 8), TABW)], tab_v)
    pltpu.sync_copy(scale_hbm.at[pl.ds(dcol, DPW)], sbbuf.at[0])
    pltpu.sync_copy(bias_hbm.at[pl.ds(dcol, DPW)], sbbuf.at[1])
    scalev = sbbuf[0]
    biasv = sbbuf[1]
    dl = lax.iota(jnp.int32, DPW)

    def chunk(c, carry):
        q0 = pl.multiple_of(c * QCH, QCH)
        pltpu.sync_copy(z_hbm.at[w, pl.ds(q0, QCH)], zbuf)
        pltpu.sync_copy(tau_hbm.at[pl.ds(pl.multiple_of(c * NBCH, 8), NBCH)],
                        taubuf)

        def row(q, carry):
            for j in range(8):
                z = zbuf[q, pl.ds(j * DPW, DPW)]
                z = _rqs_vreg(z, tab_v, dl)
                r = q * 8 + j
                tauv = plsc.load_gather(taubuf, [jnp.full((DPW,), r, jnp.int32)])
                obuf[q, pl.ds(j * DPW, DPW)] = tauv * (z * scalev + biasv)
            return carry

        lax.fori_loop(0, QCH, row, 0)
        pltpu.sync_copy(obuf, out_hbm.at[w, pl.ds(q0, QCH)])
        return carry

    lax.fori_loop(0, NCHUNK, chunk, 0)


@jax.jit
def kernel(u, tau, log_scale, bias, raw_w, raw_h, raw_s):
    raw_wT = jnp.transpose(raw_w, (0, 2, 1))
    raw_hT = jnp.transpose(raw_h, (0, 2, 1))
    raw_sT = jnp.transpose(raw_s, (0, 2, 1))
    TABF, SCALE = pl.pallas_call(
        _prep_kernel,
        out_shape=(jax.ShapeDtypeStruct((NLAYERS, NFIELD, NBINS, DIM), jnp.float32),
                   jax.ShapeDtypeStruct((1, DIM), jnp.float32)),
    )(raw_wT, raw_hT, raw_sT, log_scale.reshape(1, DIM))

    z = pl.pallas_call(
        _logit_kernel,
        grid=(B // BBZ,),
        in_specs=[pl.BlockSpec((BBZ, DIM), lambda i: (i, 0))],
        out_specs=pl.BlockSpec((BBZ, DIM), lambda i: (i, 0)),
        out_shape=jax.ShapeDtypeStruct((B, DIM), jnp.float32),
    )(u)

    tab_sc = (TABF.reshape(NLAYERS, NFIELD, NBINS, NW, DPW)
              .transpose(3, 0, 1, 2, 4).reshape(NW * TABW))
    zt = (z.reshape(B // 8, 8, NW, DPW).transpose(2, 0, 1, 3)
          .reshape(NW, B // 8, 8 * DPW))

    mesh = plsc.VectorSubcoreMesh(core_axis_name="c", subcore_axis_name="s",
                                  num_cores=NC, num_subcores=NS)
    out3 = pl.kernel(
        _sc_body,
        out_type=jax.ShapeDtypeStruct((NW, B // 8, 8 * DPW), jnp.float32),
        mesh=mesh,
        compiler_params=pltpu.CompilerParams(needs_layout_passes=False),
        scratch_types=[
            pltpu.VMEM((TABW,), jnp.float32),
            pltpu.VMEM((QCH, 8 * DPW), jnp.float32),
            pltpu.VMEM((QCH, 8 * DPW), jnp.float32),
            pltpu.VMEM((NBCH,), jnp.float32),
            pltpu.VMEM((2, DPW), jnp.float32),
        ],
    )(zt, tau.reshape(B), tab_sc, SCALE.reshape(DIM), bias)
    out = (out3.reshape(NW, B // 8, 8, DPW).transpose(1, 2, 0, 3)
           .reshape(B, DIM))
    return out


# speedup vs baseline: 2306.5170x; 2.6052x over previous
import jax
import jax.numpy as jnp
from jax import lax
from jax.experimental import pallas as pl
from jax.experimental.pallas import tpu as pltpu
from jax.experimental.pallas import tpu_sc as plsc

B = 16384
DIM = 512
NBINS = 64
BOUND = 10.0
NLAYERS = 2
EPS = 1e-6

NC = 2
NS = 16
NW = NC * NS
DPW = DIM // NW
NBCH = 1024
NCHUNK = B // NBCH
NFIELD = 6
FSTRIDE = NBINS * DPW
LSTRIDE = NFIELD * FSTRIDE
TABW = NLAYERS * LSTRIDE

BBZ = 512


def _softplus(x):
    return jnp.maximum(x, 0.0) + jnp.log(1.0 + jnp.exp(-jnp.abs(x)))


def _prep_kernel(raw_wT, raw_hT, raw_sT, log_scale, TABF, SCALE):
    ir = lax.broadcasted_iota(jnp.int32, (NBINS, NBINS), 0)
    ic = lax.broadcasted_iota(jnp.int32, (NBINS, NBINS), 1)
    L_tri = (ic <= ir).astype(jnp.float32)

    def tables(raw):
        m = jnp.max(raw, axis=0, keepdims=True)
        e = jnp.exp(raw - m)
        w = e / jnp.sum(e, axis=0, keepdims=True) * (2.0 * BOUND)
        cw = lax.dot_general(L_tri, w, (((1,), (0,)), ((), ())),
                             preferred_element_type=jnp.float32)
        return w, cw

    neg = jnp.full((1, DIM), -BOUND, jnp.float32)
    for l in range(NLAYERS):
        w, cw = tables(raw_wT[l])
        h, ch = tables(raw_hT[l])
        s = _softplus(raw_sT[l]) + 1e-4
        TABF[l, 0] = jnp.concatenate([neg, cw[: NBINS - 1] - BOUND], axis=0)
        TABF[l, 1] = 1.0 / w
        TABF[l, 2] = jnp.concatenate([neg, ch[: NBINS - 1] - BOUND], axis=0)
        TABF[l, 3] = h
        TABF[l, 4] = s[:NBINS]
        TABF[l, 5] = s[1:]
    SCALE[...] = _softplus(log_scale[...]) + 1e-4


def _logit_kernel(u, z):
    u_safe = jnp.clip(u[...], EPS, 1.0 - EPS)
    z[...] = jnp.log(u_safe / (1.0 - u_safe))


def _rqs_group(zs, tab_v, dl):
    n = len(zs)
    for l in range(NLAYERS):
        base = l * LSTRIDE
        insides = [(z > -BOUND) & (z < BOUND) for z in zs]
        zcs = [jnp.minimum(jnp.maximum(z, -BOUND), BOUND) for z in zs]
        curs = [dl + base for _ in range(n)]
        for step in (32, 16, 8, 4, 2, 1):
            cands = [c + step * DPW for c in curs]
            vals = [plsc.load_gather(tab_v, [cd]) for cd in cands]
            curs = [jnp.where(zc >= v, cd, c)
                    for zc, v, cd, c in zip(zcs, vals, cands, curs)]
        out = []
        for zc, cur, ins, z in zip(zcs, curs, insides, zs):
            xkb = plsc.load_gather(tab_v, [cur])
            invw = plsc.load_gather(tab_v, [cur + FSTRIDE])
            yb = plsc.load_gather(tab_v, [cur + 2 * FSTRIDE])
            h = plsc.load_gather(tab_v, [cur + 3 * FSTRIDE])
            d0 = plsc.load_gather(tab_v, [cur + 4 * FSTRIDE])
            d1 = plsc.load_gather(tab_v, [cur + 5 * FSTRIDE])
            s = h * invw
            xi = jnp.minimum(jnp.maximum((zc - xkb) * invw, 0.0), 1.0)
            omxi = 1.0 - xi
            num = (h * xi) * (s * xi + d0 * omxi)
            den = s + (d0 + d1 - 2.0 * s) * (xi * omxi)
            y = yb + num / den
            out.append(jnp.where(ins, y, z))
        zs = out
    return zs


RPG = DPW * 8 // DPW
QCH = NBCH // 8


def _sc_body(z_hbm, tau_hbm, tab_hbm, scale_hbm, bias_hbm, out_hbm,
             tab_v, zbuf, obuf, taubuf, sbbuf):
    w = lax.axis_index("s") * NC + lax.axis_index("c")
    dcol = pl.multiple_of(w * DPW, DPW)
    pltpu.sync_copy(tab_hbm.at[pl.ds(pl.multiple_of(w * TABW, 8), TABW)], tab_v)
    pltpu.sync_copy(scale_hbm.at[pl.ds(dcol, DPW)], sbbuf.at[0])
    pltpu.sync_copy(bias_hbm.at[pl.ds(dcol, DPW)], sbbuf.at[1])
    scalev = sbbuf[0]
    biasv = sbbuf[1]
    dl = lax.iota(jnp.int32, DPW)

    def chunk(c, carry):
        q0 = pl.multiple_of(c * QCH, QCH)
        pltpu.sync_copy(z_hbm.at[w, pl.ds(q0, QCH)], zbuf)
        pltpu.sync_copy(tau_hbm.at[pl.ds(pl.multiple_of(c * NBCH, 8), NBCH)],
                        taubuf)

        def row(q, carry):
            zs = [zbuf[q, pl.ds(j * DPW, DPW)] for j in range(8)]
            zs = _rqs_group(zs, tab_v, dl)
            for j in range(8):
                r = q * 8 + j
                tauv = plsc.load_gather(taubuf, [jnp.full((DPW,), r, jnp.int32)])
                obuf[q, pl.ds(j * DPW, DPW)] = tauv * (zs[j] * scalev + biasv)
            return carry

        lax.fori_loop(0, QCH, row, 0)
        pltpu.sync_copy(obuf, out_hbm.at[w, pl.ds(q0, QCH)])
        return carry

    lax.fori_loop(0, NCHUNK, chunk, 0)


@jax.jit
def kernel(u, tau, log_scale, bias, raw_w, raw_h, raw_s):
    raw_wT = jnp.transpose(raw_w, (0, 2, 1))
    raw_hT = jnp.transpose(raw_h, (0, 2, 1))
    raw_sT = jnp.transpose(raw_s, (0, 2, 1))
    TABF, SCALE = pl.pallas_call(
        _prep_kernel,
        out_shape=(jax.ShapeDtypeStruct((NLAYERS, NFIELD, NBINS, DIM), jnp.float32),
                   jax.ShapeDtypeStruct((1, DIM), jnp.float32)),
    )(raw_wT, raw_hT, raw_sT, log_scale.reshape(1, DIM))

    z = pl.pallas_call(
        _logit_kernel,
        grid=(B // BBZ,),
        in_specs=[pl.BlockSpec((BBZ, DIM), lambda i: (i, 0))],
        out_specs=pl.BlockSpec((BBZ, DIM), lambda i: (i, 0)),
        out_shape=jax.ShapeDtypeStruct((B, DIM), jnp.float32),
    )(u)

    tab_sc = (TABF.reshape(NLAYERS, NFIELD, NBINS, NW, DPW)
              .transpose(3, 0, 1, 2, 4).reshape(NW * TABW))
    zt = (z.reshape(B // 8, 8, NW, DPW).transpose(2, 0, 1, 3)
          .reshape(NW, B // 8, 8 * DPW))

    mesh = plsc.VectorSubcoreMesh(core_axis_name="c", subcore_axis_name="s",
                                  num_cores=NC, num_subcores=NS)
    out3 = pl.kernel(
        _sc_body,
        out_type=jax.ShapeDtypeStruct((NW, B // 8, 8 * DPW), jnp.float32),
        mesh=mesh,
        compiler_params=pltpu.CompilerParams(needs_layout_passes=False),
        scratch_types=[
            pltpu.VMEM((TABW,), jnp.float32),
            pltpu.VMEM((QCH, 8 * DPW), jnp.float32),
            pltpu.VMEM((QCH, 8 * DPW), jnp.float32),
            pltpu.VMEM((NBCH,), jnp.float32),
            pltpu.VMEM((2, DPW), jnp.float32),
        ],
    )(zt, tau.reshape(B), tab_sc, SCALE.reshape(DIM), bias)
    out = (out3.reshape(NW, B // 8, 8, DPW).transpose(1, 2, 0, 3)
           .reshape(B, DIM))
    return out
